# Initial kernel scaffold; baseline (speedup 1.0000x reference)
import jax
import jax.numpy as jnp
from jax.experimental import pallas as pl


def kernel(deltas, side_deltas, class_logits, valid_anchors, valid_indices):
    raise NotImplementedError("write your pallas kernel here")


# speedup vs baseline: 11.6729x; 11.6729x over previous
import functools

import jax
import jax.numpy as jnp
from jax import lax
from jax.experimental import pallas as pl
from jax.experimental.pallas import tpu as pltpu
from jax.experimental.pallas import tpu_sc as plsc

NMS_MAX_OUTPUTS = 2000
IOU_THRESH = 0.3
SCORE_THRESH = 0.7
_NEG_INF = float("-inf")


def _sc_gather(table, idx):
    info = plsc.get_sparse_core_info()
    nw = info.num_cores * info.num_subcores
    m = idx.shape[0]
    ipw = m // nw
    cpw = ipw // 64
    mesh = plsc.VectorSubcoreMesh(core_axis_name="c", subcore_axis_name="s")

    @functools.partial(
        pl.kernel,
        out_type=jax.ShapeDtypeStruct((m, 8), jnp.float32),
        mesh=mesh,
        compiler_params=pltpu.CompilerParams(use_tc_tiling_on_sc=False),
        scratch_types=[
            pltpu.VMEM((ipw,), jnp.int32),
            pltpu.VMEM((64, 8), jnp.float32),
            pltpu.SemaphoreType.DMA,
        ],
    )
    def k(table_hbm, idx_hbm, out_hbm, idx_v, rows_v, sem):
        wid = lax.axis_index("s") * info.num_cores + lax.axis_index("c")
        base = wid * ipw
        pltpu.sync_copy(idx_hbm.at[pl.ds(base, ipw)], idx_v)
        for i in range(cpw):
            pltpu.async_copy(
                table_hbm.at[idx_v.at[pl.ds(i * 64, 64)]], rows_v, sem).wait()
            pltpu.sync_copy(rows_v, out_hbm.at[pl.ds(base + i * 64, 64)])

    return k(table, idx)


def _tc_nms_body(g_ref, anc_ref, out_ref, *, a_valid, npc, max_out):
    out_ref[...] = jnp.zeros_like(out_ref)

    d0 = g_ref[0, 0]
    d1 = g_ref[0, 1]
    cl0 = g_ref[0, 2]
    cl1 = g_ref[0, 3]
    ax1 = anc_ref[0]
    ay1 = anc_ref[1]
    ax2 = anc_ref[2]
    ay2 = anc_ref[3]

    mcl = jnp.maximum(cl0, cl1)
    e0 = jnp.exp(cl0 - mcl)
    e1 = jnp.exp(cl1 - mcl)
    fg = e1 / (e0 + e1)

    h_a = ay2 - ay1
    cy = (ay1 + ay2) * 0.5 + (d0 * 0.1) * h_a
    h_n = h_a * jnp.exp(d1 * 0.2)
    x1 = ax1
    y1 = cy - h_n * 0.5
    x2 = ax2
    y2 = cy + h_n * 0.5
    area = jnp.maximum(x2 - x1, 0.0) * jnp.maximum(y2 - y1, 0.0)

    ids = (lax.broadcasted_iota(jnp.int32, (8, npc), 0) * npc
           + lax.broadcasted_iota(jnp.int32, (8, npc), 1))
    valid = ids < a_valid
    s0 = jnp.where(valid & (fg >= SCORE_THRESH), fg, _NEG_INF)

    lane = lax.broadcasted_iota(jnp.int32, (1, 8), 1)

    def cond(carry):
        k, m, _ = carry
        return (k < max_out) & (m > _NEG_INF)

    def body(carry):
        k, m, s = carry
        bid = jnp.min(jnp.where(s == m, ids, jnp.int32(2 ** 30)))
        bmask = ids == bid

        def ext(v):
            return jnp.max(jnp.where(bmask, v, _NEG_INF))

        bx1 = ext(x1)
        by1 = ext(y1)
        bx2 = ext(x2)
        by2 = ext(y2)
        bc0 = ext(cl0)
        bc1 = ext(cl1)

        xx1 = jnp.maximum(bx1, x1)
        yy1 = jnp.maximum(by1, y1)
        xx2 = jnp.minimum(bx2, x2)
        yy2 = jnp.minimum(by2, y2)
        inter = jnp.maximum(xx2 - xx1, 0.0) * jnp.maximum(yy2 - yy1, 0.0)
        area_b = jnp.maximum(bx2 - bx1, 0.0) * jnp.maximum(by2 - by1, 0.0)
        iou = inter / (area + area_b - inter + 1e-8)
        s2 = jnp.where((iou > IOU_THRESH) | bmask, _NEG_INF, s)

        row = jnp.where(lane == 0, bx1,
              jnp.where(lane == 1, by1,
              jnp.where(lane == 2, bx2,
              jnp.where(lane == 3, by2,
              jnp.where(lane == 4, m,
              jnp.where(lane == 5, bc0,
              jnp.where(lane == 6, bc1, 1.0)))))))
        out_ref[0, pl.ds(k, 1), :] = row
        return k + 1, jnp.max(s2), s2

    lax.while_loop(cond, body, (jnp.int32(0), jnp.max(s0), s0))


def _tc_nms(g4, anc4, b, a_valid, npc, max_out):
    body = functools.partial(_tc_nms_body, a_valid=a_valid, npc=npc,
                             max_out=max_out)
    return pl.pallas_call(
        body,
        grid=(b,),
        in_specs=[
            pl.BlockSpec((1, 4, 8, npc), lambda i: (i, 0, 0, 0)),
            pl.BlockSpec((4, 8, npc), lambda i: (0, 0, 0)),
        ],
        out_specs=pl.BlockSpec((1, max_out, 8), lambda i: (i, 0, 0)),
        out_shape=jax.ShapeDtypeStruct((b, max_out, 8), jnp.float32),
    )(g4, anc4)


def kernel(deltas, side_deltas, class_logits, valid_anchors, valid_indices):
    del side_deltas
    b, n, _ = deltas.shape
    a = valid_anchors.shape[0]
    a_pad = ((a + 1023) // 1024) * 1024
    npc = a_pad // 8

    table = jnp.concatenate(
        [deltas.reshape(b * n, 2), class_logits.reshape(b * n, 2),
         jnp.zeros((b * n, 4), jnp.float32)], axis=1)

    vi = jnp.concatenate(
        [valid_indices.astype(jnp.int32), jnp.zeros((a_pad - a,), jnp.int32)])
    flat_idx = (vi[None, :] + (jnp.arange(b, dtype=jnp.int32) * n)[:, None])

    gathered = _sc_gather(table, flat_idx.reshape(b * a_pad))
    g4 = gathered.reshape(b, a_pad, 8).transpose(0, 2, 1)[:, :4]
    g4 = g4.reshape(b, 4, 8, npc)

    anc = jnp.concatenate(
        [valid_anchors, jnp.zeros((a_pad - a, 4), jnp.float32)], axis=0)
    anc4 = anc.T.reshape(4, 8, npc)

    res = _tc_nms(g4, anc4, b, a, npc, NMS_MAX_OUTPUTS)
    tag = res[..., 7:8]
    out_boxes = jnp.concatenate([res[..., 0:4], tag], axis=-1)
    out_scores = jnp.concatenate([res[..., 4:5], tag], axis=-1)
    out_logits = jnp.concatenate([res[..., 5:7], tag], axis=-1)
    return out_boxes, out_scores, out_logits


# speedup vs baseline: 13.0699x; 1.1197x over previous
import functools

import jax
import jax.numpy as jnp
from jax import lax
from jax.experimental import pallas as pl
from jax.experimental.pallas import tpu as pltpu
from jax.experimental.pallas import tpu_sc as plsc

NMS_MAX_OUTPUTS = 2000
IOU_THRESH = 0.3
SCORE_THRESH = 0.7
_NEG_INF = float("-inf")


def _sc_gather(table, idx):
    info = plsc.get_sparse_core_info()
    nw = info.num_cores * info.num_subcores
    m = idx.shape[0]
    ipw = m // nw
    cpw = ipw // 64
    mesh = plsc.VectorSubcoreMesh(core_axis_name="c", subcore_axis_name="s")

    @functools.partial(
        pl.kernel,
        out_type=jax.ShapeDtypeStruct((m, 8), jnp.float32),
        mesh=mesh,
        compiler_params=pltpu.CompilerParams(use_tc_tiling_on_sc=False),
        scratch_types=[
            pltpu.VMEM((ipw,), jnp.int32),
            pltpu.VMEM((64, 8), jnp.float32),
            pltpu.SemaphoreType.DMA,
        ],
    )
    def k(table_hbm, idx_hbm, out_hbm, idx_v, rows_v, sem):
        wid = lax.axis_index("s") * info.num_cores + lax.axis_index("c")
        base = wid * ipw
        pltpu.sync_copy(idx_hbm.at[pl.ds(base, ipw)], idx_v)
        for i in range(cpw):
            pltpu.async_copy(
                table_hbm.at[idx_v.at[pl.ds(i * 64, 64)]], rows_v, sem).wait()
            pltpu.sync_copy(rows_v, out_hbm.at[pl.ds(base + i * 64, 64)])

    return k(table, idx)


def _tc_nms_body(g_ref, anc_ref, out_ref, *, b, a_valid, npc, max_out):
    out_ref[...] = jnp.zeros_like(out_ref)

    ax1 = anc_ref[0]
    ay1 = anc_ref[1]
    ax2 = anc_ref[2]
    ay2 = anc_ref[3]
    h_a = ay2 - ay1
    cy_a = (ay1 + ay2) * 0.5

    ids = (lax.broadcasted_iota(jnp.int32, (8, npc), 0) * npc
           + lax.broadcasted_iota(jnp.int32, (8, npc), 1))
    valid = ids < a_valid
    lane = lax.broadcasted_iota(jnp.int32, (1, 8), 1)

    per = []
    for i in range(b):
        d0 = g_ref[i, 0]
        d1 = g_ref[i, 1]
        cl0 = g_ref[i, 2]
        cl1 = g_ref[i, 3]
        mcl = jnp.maximum(cl0, cl1)
        e0 = jnp.exp(cl0 - mcl)
        e1 = jnp.exp(cl1 - mcl)
        fg = e1 / (e0 + e1)
        cy = cy_a + (d0 * 0.1) * h_a
        h_n = h_a * jnp.exp(d1 * 0.2)
        y1 = cy - h_n * 0.5
        y2 = cy + h_n * 0.5
        area = jnp.maximum(ax2 - ax1, 0.0) * jnp.maximum(y2 - y1, 0.0)
        s0 = jnp.where(valid & (fg >= SCORE_THRESH), fg, _NEG_INF)
        per.append((s0, y1, y2, area))

    def step(i, k, m, s):
        _, y1, y2, area = per[i]
        active = (k < max_out) & (m > _NEG_INF)
        bid = jnp.min(jnp.where(s == m, ids, jnp.int32(2 ** 30)))
        bmask = ids == bid

        def ext(v):
            return jnp.max(jnp.where(bmask, v, _NEG_INF))

        bx1 = ext(ax1)
        by1 = ext(y1)
        bx2 = ext(ax2)
        by2 = ext(y2)
        bc0 = ext(g_ref[i, 2])
        bc1 = ext(g_ref[i, 3])

        xx1 = jnp.maximum(bx1, ax1)
        yy1 = jnp.maximum(by1, y1)
        xx2 = jnp.minimum(bx2, ax2)
        yy2 = jnp.minimum(by2, y2)
        inter = jnp.maximum(xx2 - xx1, 0.0) * jnp.maximum(yy2 - yy1, 0.0)
        area_b = jnp.maximum(bx2 - bx1, 0.0) * jnp.maximum(by2 - by1, 0.0)
        iou = inter / (area + area_b - inter + 1e-8)
        s2 = jnp.where((iou > IOU_THRESH) | bmask, _NEG_INF, s)
        s_new = jnp.where(active, s2, s)

        row = jnp.where(lane == 0, bx1,
              jnp.where(lane == 1, by1,
              jnp.where(lane == 2, bx2,
              jnp.where(lane == 3, by2,
              jnp.where(lane == 4, m,
              jnp.where(lane == 5, bc0,
              jnp.where(lane == 6, bc1, 1.0)))))))

        @pl.when(active)
        def _():
            out_ref[i, pl.ds(k, 1), :] = row

        return (k + active.astype(jnp.int32), jnp.max(s_new), s_new)

    def cond(carry):
        alive = [(c[0] < max_out) & (c[1] > _NEG_INF) for c in carry]
        out = alive[0]
        for a in alive[1:]:
            out = out | a
        return out

    def body(carry):
        return tuple(step(i, *carry[i]) for i in range(b))

    init = tuple((jnp.int32(0), jnp.max(per[i][0]), per[i][0])
                 for i in range(b))
    lax.while_loop(cond, body, init)


def _tc_nms(g4, anc4, b, a_valid, npc, max_out):
    body = functools.partial(_tc_nms_body, b=b, a_valid=a_valid, npc=npc,
                             max_out=max_out)
    return pl.pallas_call(
        body,
        out_shape=jax.ShapeDtypeStruct((b, max_out, 8), jnp.float32),
    )(g4, anc4)


def kernel(deltas, side_deltas, class_logits, valid_anchors, valid_indices):
    del side_deltas
    b, n, _ = deltas.shape
    a = valid_anchors.shape[0]
    a_pad = ((a + 1023) // 1024) * 1024
    npc = a_pad // 8

    table = jnp.concatenate(
        [deltas.reshape(b * n, 2), class_logits.reshape(b * n, 2),
         jnp.zeros((b * n, 4), jnp.float32)], axis=1)

    vi = jnp.concatenate(
        [valid_indices.astype(jnp.int32), jnp.zeros((a_pad - a,), jnp.int32)])
    flat_idx = (vi[None, :] + (jnp.arange(b, dtype=jnp.int32) * n)[:, None])

    gathered = _sc_gather(table, flat_idx.reshape(b * a_pad))
    g4 = gathered.reshape(b, a_pad, 8).transpose(0, 2, 1)[:, :4]
    g4 = g4.reshape(b, 4, 8, npc)

    anc = jnp.concatenate(
        [valid_anchors, jnp.zeros((a_pad - a, 4), jnp.float32)], axis=0)
    anc4 = anc.T.reshape(4, 8, npc)

    res = _tc_nms(g4, anc4, b, a, npc, NMS_MAX_OUTPUTS)
    tag = res[..., 7:8]
    out_boxes = jnp.concatenate([res[..., 0:4], tag], axis=-1)
    out_scores = jnp.concatenate([res[..., 4:5], tag], axis=-1)
    out_logits = jnp.concatenate([res[..., 5:7], tag], axis=-1)
    return out_boxes, out_scores, out_logits
